# baseline (speedup 1.0000x reference)
import functools

import jax
import jax.numpy as jnp
from jax.experimental import pallas as pl
from jax.experimental.pallas import tpu as pltpu

_MATCH_THRESH = 0.5
_NEG_POS = 3
_V0 = 0.1
_V1 = 0.2
_A = 16320
_A_PAD = 16384
_R = 128
_L = 128
_G = 50
_C = 21


def _tree(vals, op):
    while len(vals) > 1:
        nxt = [op(vals[i], vals[i + 1]) for i in range(0, len(vals) - 1, 2)]
        if len(vals) % 2:
            nxt.append(vals[-1])
        vals = nxt
    return vals[0]


def _gather(tab, idx):
    tab2d = jnp.broadcast_to(tab[None, :], (_R, _L))
    return jnp.take_along_axis(tab2d, idx, axis=1)


def _match_and_loss(pcx, pcy, pw, ph, logits, locpred, gtb_ref, gtbt_ref,
                    labt_ref, use_labels, fiota, valid):
    px1 = pcx - pw * 0.5
    py1 = pcy - ph * 0.5
    px2 = pcx + pw * 0.5
    py2 = pcy + ph * 0.5
    wb = px2 - px1
    hb = py2 - py1
    area_b = wb * hb

    zero = jnp.zeros((_R, _L), jnp.float32)
    btv = zero
    bg = jnp.zeros((_R, _L), jnp.int32)

    rowidx = []
    for g in range(_G):
        gx1 = gtb_ref[0, g, 0]
        gy1 = gtb_ref[0, g, 1]
        gx2 = gtb_ref[0, g, 2]
        gy2 = gtb_ref[0, g, 3]
        ixmin = jnp.maximum(px1, gx1)
        iymin = jnp.maximum(py1, gy1)
        ixmax = jnp.minimum(px2, gx2)
        iymax = jnp.minimum(py2, gy2)
        iw = jnp.clip(ixmax - ixmin, 0.0, None)
        ih = jnp.clip(iymax - iymin, 0.0, None)
        inter = iw * ih
        area_a = (gx2 - gx1) * (gy2 - gy1)
        union = area_a + area_b - inter
        iou = inter / union
        upd = iou > btv
        btv = jnp.where(upd, iou, btv)
        bg = jnp.where(upd, g, bg)
        mval = jnp.max(iou, keepdims=True)
        rowidx.append(jnp.min(jnp.where(iou == mval, fiota, _A_PAD),
                              keepdims=True))

    fg = _tree([jnp.where(fiota == rowidx[g], g, -1) for g in range(_G)],
               jnp.maximum)
    forced = fg >= 0
    btv = jnp.where(forced, 2.0, btv)
    bg = jnp.where(forced, fg, bg)

    m1 = _gather(gtbt_ref[0, 0], bg)
    m2 = _gather(gtbt_ref[0, 1], bg)
    m3 = _gather(gtbt_ref[0, 2], bg)
    m4 = _gather(gtbt_ref[0, 3], bg)
    if use_labels:
        lab = _gather(labt_ref[0, 0] + 1, bg)
    else:
        lab = jnp.ones((_R, _L), jnp.int32)

    conf = jnp.where(btv < _MATCH_THRESH, 0, lab)
    pos = conf > 0
    nposi = jnp.sum(pos.astype(jnp.int32), keepdims=True)

    gcx = ((m1 + m3) * 0.5 - pcx) / (_V0 * pw)
    gcy = ((m2 + m4) * 0.5 - pcy) / (_V0 * ph)
    gw = jnp.log(jnp.maximum((m3 - m1) / pw, 1e-8)) / _V1
    gh = jnp.log(jnp.maximum((m4 - m2) / ph, 1e-8)) / _V1
    loc_loss = jnp.zeros((1, 1), jnp.float32)
    for pred, tgt in zip(locpred, (gcx, gcy, gw, gh)):
        d = pred - tgt
        ad = jnp.abs(d)
        hub = jnp.where(ad < 1.0, 0.5 * d * d, ad - 0.5)
        loc_loss = loc_loss + jnp.sum(jnp.where(pos, hub, 0.0),
                                      keepdims=True)

    mx = logits[0]
    for lg_ in logits[1:]:
        mx = jnp.maximum(mx, lg_)
    s = jnp.exp(logits[0] - mx)
    for lg_ in logits[1:]:
        s = s + jnp.exp(lg_ - mx)
    lse = mx + jnp.log(s)
    sel = logits[0]
    for c in range(1, len(logits)):
        sel = jnp.where(conf == c, logits[c], sel)
    ce = lse - sel

    pos_loss = jnp.sum(jnp.where(pos, ce, 0.0), keepdims=True)

    neg = (conf == 0) & valid
    negcnt = jnp.sum(neg.astype(jnp.int32), keepdims=True)
    negnum = jnp.maximum(10, jnp.minimum(nposi * _NEG_POS, _A - nposi))
    k = jnp.minimum(negnum, negcnt)
    x = jnp.where(neg, jax.lax.bitcast_convert_type(ce, jnp.int32),
                  jnp.int32(-1))

    return pos_loss, loc_loss, nposi.astype(jnp.float32), x, k, ce


def _count_ge(x, piv):
    return jnp.sum((x >= piv).astype(jnp.int32), keepdims=True)


def _neg_loss2(xa, ka, cea, xo, ko, ceo):
    pa = jnp.zeros((1, 1), jnp.int32)
    po = jnp.zeros((1, 1), jnp.int32)
    piv = jnp.full((1, 1), 1 << 30, jnp.int32)
    pa = jnp.where(_count_ge(xa, piv) >= ka, piv, pa)
    po = jnp.where(_count_ge(xo, piv) >= ko, piv, po)
    for s in range(28, -1, -2):
        ca1 = _count_ge(xa, pa + (1 << s)) >= ka
        ca2 = _count_ge(xa, pa + (2 << s)) >= ka
        ca3 = _count_ge(xa, pa + (3 << s)) >= ka
        co1 = _count_ge(xo, po + (1 << s)) >= ko
        co2 = _count_ge(xo, po + (2 << s)) >= ko
        co3 = _count_ge(xo, po + (3 << s)) >= ko
        ba = (ca1.astype(jnp.int32) + ca2.astype(jnp.int32)
              + ca3.astype(jnp.int32))
        bo = (co1.astype(jnp.int32) + co2.astype(jnp.int32)
              + co3.astype(jnp.int32))
        pa = pa + (ba << s)
        po = po + (bo << s)

    def finish(p, x, ce, k):
        v = jax.lax.bitcast_convert_type(p, jnp.float32)
        gt_mask = x > p
        cnt_gt = jnp.sum(gt_mask.astype(jnp.int32), keepdims=True)
        sum_gt = jnp.sum(jnp.where(gt_mask, ce, 0.0), keepdims=True)
        nl = sum_gt + (k - cnt_gt).astype(jnp.float32) * v
        return jnp.where(k > 0, nl, 0.0)

    return finish(pa, xa, cea, ka), finish(po, xo, ceo, ko)


def _body(obj_ref, rl_ref, pc_ref, plc_ref, an_ref, gtb_ref, gtbt_ref,
          labt_ref, out_ref):
    b = pl.program_id(0)

    fiota = (jax.lax.broadcasted_iota(jnp.int32, (_R, _L), 0) * _L
             + jax.lax.broadcasted_iota(jnp.int32, (_R, _L), 1))
    valid = fiota < _A

    acx = an_ref[0]
    acy = an_ref[1]
    aw = an_ref[2]
    ah = an_ref[3]
    rl = [rl_ref[0, i] for i in range(4)]

    arm_pos, arm_loc, arm_n, xa, ka, cea = _match_and_loss(
        acx, acy, aw, ah,
        [obj_ref[0, 0], obj_ref[0, 1]],
        rl, gtb_ref, gtbt_ref, labt_ref, False, fiota, valid)

    ocx = acx + rl[0] * _V0 * aw
    ocy = acy + rl[1] * _V0 * ah
    ow = aw * jnp.exp(rl[2] * _V1)
    oh = ah * jnp.exp(rl[3] * _V1)
    odm_pos, odm_loc, odm_n, xo, ko, ceo = _match_and_loss(
        ocx, ocy, ow, oh,
        [pc_ref[0, c] for c in range(_C)],
        [plc_ref[0, i] for i in range(4)],
        gtb_ref, gtbt_ref, labt_ref, True, fiota, valid)

    neg_a, neg_o = _neg_loss2(xa, ka, cea, xo, ko, ceo)
    arm_cls = arm_pos + neg_a
    odm_cls = odm_pos + neg_o

    lane = jax.lax.broadcasted_iota(jnp.int32, (1, _L), 1)
    row = (jnp.where(lane == 0, arm_cls, 0.0)
           + jnp.where(lane == 1, arm_loc, 0.0)
           + jnp.where(lane == 2, arm_n, 0.0)
           + jnp.where(lane == 3, odm_cls, 0.0)
           + jnp.where(lane == 4, odm_loc, 0.0)
           + jnp.where(lane == 5, odm_n, 0.0))

    out_ref[...] = row[None]


def _prep(x):
    xt = jnp.transpose(x, (0, 2, 1))
    xt = jnp.pad(xt, ((0, 0), (0, 0), (0, _A_PAD - _A)))
    return xt.reshape(x.shape[0], x.shape[2], _R, _L)


@jax.jit
def kernel(objectness, refine_loc, pred_conf, pred_loc, anchors, gt_boxes,
           gt_labels):
    B = objectness.shape[0]
    obj_t = _prep(objectness)
    rl_t = _prep(refine_loc)
    pc_t = _prep(pred_conf)
    plc_t = _prep(pred_loc)
    an_t = _prep(anchors[:1])[0]

    out = pl.pallas_call(
        _body,
        grid=(B,),
        in_specs=[
            pl.BlockSpec((1, 2, _R, _L), lambda b: (b, 0, 0, 0)),
            pl.BlockSpec((1, 4, _R, _L), lambda b: (b, 0, 0, 0)),
            pl.BlockSpec((1, _C, _R, _L), lambda b: (b, 0, 0, 0)),
            pl.BlockSpec((1, 4, _R, _L), lambda b: (b, 0, 0, 0)),
            pl.BlockSpec((4, _R, _L), lambda b: (0, 0, 0)),
            pl.BlockSpec((1, _G, 4), lambda b: (b, 0, 0),
                         memory_space=pltpu.SMEM),
            pl.BlockSpec((1, 4, _L), lambda b: (b, 0, 0)),
            pl.BlockSpec((1, 1, _L), lambda b: (b, 0, 0)),
        ],
        out_specs=pl.BlockSpec((1, 1, _L), lambda b: (b, 0, 0)),
        out_shape=jax.ShapeDtypeStruct((B, 1, _L), jnp.float32),
        compiler_params=pltpu.CompilerParams(
            dimension_semantics=("parallel",)),
    )(obj_t, rl_t, pc_t, plc_t, an_t, gt_boxes,
      jnp.pad(jnp.transpose(gt_boxes, (0, 2, 1)),
              ((0, 0), (0, 0), (0, _L - _G))),
      jnp.pad(gt_labels.reshape(B, 1, _G),
              ((0, 0), (0, 0), (0, _L - _G))))

    r = jnp.sum(out[:, 0, :], axis=0)
    arm_cls = r[0] / r[2]
    arm_loc = r[1] / r[2]
    odm_cls = r[3] / r[5]
    odm_loc = r[4] / r[5]
    total = arm_cls + arm_loc + odm_cls + odm_loc
    return (total, odm_cls, odm_loc, arm_cls, arm_loc)


# speedup vs baseline: 1.0031x; 1.0031x over previous
import functools

import jax
import jax.numpy as jnp
from jax.experimental import pallas as pl
from jax.experimental.pallas import tpu as pltpu

_MATCH_THRESH = 0.5
_NEG_POS = 3
_V0 = 0.1
_V1 = 0.2
_A = 16320
_A_PAD = 16384
_R = 128
_L = 128
_G = 50
_C = 21


def _tree(vals, op):
    while len(vals) > 1:
        nxt = [op(vals[i], vals[i + 1]) for i in range(0, len(vals) - 1, 2)]
        if len(vals) % 2:
            nxt.append(vals[-1])
        vals = nxt
    return vals[0]


def _gather(tab, idx):
    tab2d = jnp.broadcast_to(tab[None, :], (_R, _L))
    return jnp.take_along_axis(tab2d, idx, axis=1)


def _match(priors_sets, gtb_ref, fiota):
    n = len(priors_sets)
    pf = []
    for pcx, pcy, pw, ph in priors_sets:
        px1 = pcx - pw * 0.5
        py1 = pcy - ph * 0.5
        px2 = pcx + pw * 0.5
        py2 = pcy + ph * 0.5
        area_b = (px2 - px1) * (py2 - py1)
        pf.append((px1, py1, px2, py2, area_b))

    zero = jnp.zeros((_R, _L), jnp.float32)
    btv = [zero] * n
    bg = [jnp.zeros((_R, _L), jnp.int32)] * n
    rowidx = [[] for _ in range(n)]

    for g in range(_G):
        gx1 = gtb_ref[0, g, 0]
        gy1 = gtb_ref[0, g, 1]
        gx2 = gtb_ref[0, g, 2]
        gy2 = gtb_ref[0, g, 3]
        area_a = (gx2 - gx1) * (gy2 - gy1)
        for p in range(n):
            px1, py1, px2, py2, area_b = pf[p]
            ixmin = jnp.maximum(px1, gx1)
            iymin = jnp.maximum(py1, gy1)
            ixmax = jnp.minimum(px2, gx2)
            iymax = jnp.minimum(py2, gy2)
            iw = jnp.clip(ixmax - ixmin, 0.0, None)
            ih = jnp.clip(iymax - iymin, 0.0, None)
            inter = iw * ih
            union = area_a + area_b - inter
            iou = inter / union
            upd = iou > btv[p]
            btv[p] = jnp.where(upd, iou, btv[p])
            bg[p] = jnp.where(upd, g, bg[p])
            mval = jnp.max(iou, keepdims=True)
            rowidx[p].append(jnp.min(
                jnp.where(iou == mval, fiota, _A_PAD), keepdims=True))

    out = []
    for p in range(n):
        fg = _tree([jnp.where(fiota == rowidx[p][g], g, -1)
                    for g in range(_G)], jnp.maximum)
        forced = fg >= 0
        out.append((jnp.where(forced, 2.0, btv[p]),
                    jnp.where(forced, fg, bg[p])))
    return out


def _match_and_loss(btv, bg, pcx, pcy, pw, ph, logits, locpred, gtbt_ref,
                    labt_ref, use_labels, fiota, valid):
    m1 = _gather(gtbt_ref[0, 0], bg)
    m2 = _gather(gtbt_ref[0, 1], bg)
    m3 = _gather(gtbt_ref[0, 2], bg)
    m4 = _gather(gtbt_ref[0, 3], bg)
    if use_labels:
        lab = _gather(labt_ref[0, 0] + 1, bg)
    else:
        lab = jnp.ones((_R, _L), jnp.int32)

    conf = jnp.where(btv < _MATCH_THRESH, 0, lab)
    pos = conf > 0
    nposi = jnp.sum(pos.astype(jnp.int32), keepdims=True)

    gcx = ((m1 + m3) * 0.5 - pcx) / (_V0 * pw)
    gcy = ((m2 + m4) * 0.5 - pcy) / (_V0 * ph)
    gw = jnp.log(jnp.maximum((m3 - m1) / pw, 1e-8)) / _V1
    gh = jnp.log(jnp.maximum((m4 - m2) / ph, 1e-8)) / _V1
    loc_loss = jnp.zeros((1, 1), jnp.float32)
    for pred, tgt in zip(locpred, (gcx, gcy, gw, gh)):
        d = pred - tgt
        ad = jnp.abs(d)
        hub = jnp.where(ad < 1.0, 0.5 * d * d, ad - 0.5)
        loc_loss = loc_loss + jnp.sum(jnp.where(pos, hub, 0.0),
                                      keepdims=True)

    mx = logits[0]
    for lg_ in logits[1:]:
        mx = jnp.maximum(mx, lg_)
    s = jnp.exp(logits[0] - mx)
    for lg_ in logits[1:]:
        s = s + jnp.exp(lg_ - mx)
    lse = mx + jnp.log(s)
    sel = logits[0]
    for c in range(1, len(logits)):
        sel = jnp.where(conf == c, logits[c], sel)
    ce = lse - sel

    pos_loss = jnp.sum(jnp.where(pos, ce, 0.0), keepdims=True)

    neg = (conf == 0) & valid
    negcnt = jnp.sum(neg.astype(jnp.int32), keepdims=True)
    negnum = jnp.maximum(10, jnp.minimum(nposi * _NEG_POS, _A - nposi))
    k = jnp.minimum(negnum, negcnt)
    x = jnp.where(neg, jax.lax.bitcast_convert_type(ce, jnp.int32),
                  jnp.int32(-1))

    return pos_loss, loc_loss, nposi.astype(jnp.float32), x, k, ce


def _count_ge(x, piv):
    return jnp.sum((x >= piv).astype(jnp.int32), keepdims=True)


def _neg_loss2(xa, ka, cea, xo, ko, ceo):
    pa = jnp.zeros((1, 1), jnp.int32)
    po = jnp.zeros((1, 1), jnp.int32)
    piv = jnp.full((1, 1), 1 << 30, jnp.int32)
    pa = jnp.where(_count_ge(xa, piv) >= ka, piv, pa)
    po = jnp.where(_count_ge(xo, piv) >= ko, piv, po)
    for s in range(28, -1, -2):
        ca1 = _count_ge(xa, pa + (1 << s)) >= ka
        ca2 = _count_ge(xa, pa + (2 << s)) >= ka
        ca3 = _count_ge(xa, pa + (3 << s)) >= ka
        co1 = _count_ge(xo, po + (1 << s)) >= ko
        co2 = _count_ge(xo, po + (2 << s)) >= ko
        co3 = _count_ge(xo, po + (3 << s)) >= ko
        ba = (ca1.astype(jnp.int32) + ca2.astype(jnp.int32)
              + ca3.astype(jnp.int32))
        bo = (co1.astype(jnp.int32) + co2.astype(jnp.int32)
              + co3.astype(jnp.int32))
        pa = pa + (ba << s)
        po = po + (bo << s)

    def finish(p, x, ce, k):
        v = jax.lax.bitcast_convert_type(p, jnp.float32)
        gt_mask = x > p
        cnt_gt = jnp.sum(gt_mask.astype(jnp.int32), keepdims=True)
        sum_gt = jnp.sum(jnp.where(gt_mask, ce, 0.0), keepdims=True)
        nl = sum_gt + (k - cnt_gt).astype(jnp.float32) * v
        return jnp.where(k > 0, nl, 0.0)

    return finish(pa, xa, cea, ka), finish(po, xo, ceo, ko)


def _body(obj_ref, rl_ref, pc_ref, plc_ref, an_ref, gtb_ref, gtbt_ref,
          labt_ref, out_ref):
    b = pl.program_id(0)

    fiota = (jax.lax.broadcasted_iota(jnp.int32, (_R, _L), 0) * _L
             + jax.lax.broadcasted_iota(jnp.int32, (_R, _L), 1))
    valid = fiota < _A

    acx = an_ref[0]
    acy = an_ref[1]
    aw = an_ref[2]
    ah = an_ref[3]
    rl = [rl_ref[0, i] for i in range(4)]

    ocx = acx + rl[0] * _V0 * aw
    ocy = acy + rl[1] * _V0 * ah
    ow = aw * jnp.exp(rl[2] * _V1)
    oh = ah * jnp.exp(rl[3] * _V1)

    (btv_a, bg_a), (btv_o, bg_o) = _match(
        [(acx, acy, aw, ah), (ocx, ocy, ow, oh)], gtb_ref, fiota)

    arm_pos, arm_loc, arm_n, xa, ka, cea = _match_and_loss(
        btv_a, bg_a, acx, acy, aw, ah,
        [obj_ref[0, 0], obj_ref[0, 1]],
        rl, gtbt_ref, labt_ref, False, fiota, valid)

    odm_pos, odm_loc, odm_n, xo, ko, ceo = _match_and_loss(
        btv_o, bg_o, ocx, ocy, ow, oh,
        [pc_ref[0, c] for c in range(_C)],
        [plc_ref[0, i] for i in range(4)],
        gtbt_ref, labt_ref, True, fiota, valid)

    neg_a, neg_o = _neg_loss2(xa, ka, cea, xo, ko, ceo)
    arm_cls = arm_pos + neg_a
    odm_cls = odm_pos + neg_o

    lane = jax.lax.broadcasted_iota(jnp.int32, (1, _L), 1)
    row = (jnp.where(lane == 0, arm_cls, 0.0)
           + jnp.where(lane == 1, arm_loc, 0.0)
           + jnp.where(lane == 2, arm_n, 0.0)
           + jnp.where(lane == 3, odm_cls, 0.0)
           + jnp.where(lane == 4, odm_loc, 0.0)
           + jnp.where(lane == 5, odm_n, 0.0))

    out_ref[...] = row[None]


def _prep(x):
    xt = jnp.transpose(x, (0, 2, 1))
    xt = jnp.pad(xt, ((0, 0), (0, 0), (0, _A_PAD - _A)))
    return xt.reshape(x.shape[0], x.shape[2], _R, _L)


@jax.jit
def kernel(objectness, refine_loc, pred_conf, pred_loc, anchors, gt_boxes,
           gt_labels):
    B = objectness.shape[0]
    obj_t = _prep(objectness)
    rl_t = _prep(refine_loc)
    pc_t = _prep(pred_conf)
    plc_t = _prep(pred_loc)
    an_t = _prep(anchors[:1])[0]

    out = pl.pallas_call(
        _body,
        grid=(B,),
        in_specs=[
            pl.BlockSpec((1, 2, _R, _L), lambda b: (b, 0, 0, 0)),
            pl.BlockSpec((1, 4, _R, _L), lambda b: (b, 0, 0, 0)),
            pl.BlockSpec((1, _C, _R, _L), lambda b: (b, 0, 0, 0)),
            pl.BlockSpec((1, 4, _R, _L), lambda b: (b, 0, 0, 0)),
            pl.BlockSpec((4, _R, _L), lambda b: (0, 0, 0)),
            pl.BlockSpec((1, _G, 4), lambda b: (b, 0, 0),
                         memory_space=pltpu.SMEM),
            pl.BlockSpec((1, 4, _L), lambda b: (b, 0, 0)),
            pl.BlockSpec((1, 1, _L), lambda b: (b, 0, 0)),
        ],
        out_specs=pl.BlockSpec((1, 1, _L), lambda b: (b, 0, 0)),
        out_shape=jax.ShapeDtypeStruct((B, 1, _L), jnp.float32),
        compiler_params=pltpu.CompilerParams(
            dimension_semantics=("parallel",)),
    )(obj_t, rl_t, pc_t, plc_t, an_t, gt_boxes,
      jnp.pad(jnp.transpose(gt_boxes, (0, 2, 1)),
              ((0, 0), (0, 0), (0, _L - _G))),
      jnp.pad(gt_labels.reshape(B, 1, _G),
              ((0, 0), (0, 0), (0, _L - _G))))

    r = jnp.sum(out[:, 0, :], axis=0)
    arm_cls = r[0] / r[2]
    arm_loc = r[1] / r[2]
    odm_cls = r[3] / r[5]
    odm_loc = r[4] / r[5]
    total = arm_cls + arm_loc + odm_cls + odm_loc
    return (total, odm_cls, odm_loc, arm_cls, arm_loc)
